# baseline (speedup 1.0000x reference)
import functools

import jax
import jax.numpy as jnp
from jax import lax
from jax.experimental import pallas as pl
from jax.experimental.pallas import tpu as pltpu
from jax.experimental.pallas import tpu_sc as plsc

N_GENES_K = 20000
N_TFS_K = 2048
N_CONN_K = 65536
BATCH_K = 1024
_HALF_B = BATCH_K // 2

_NC = 2
_NS = 16
_NW = _NC * _NS
_EDGE_SPLIT = 4
_ROW_GROUPS = _NW // _EDGE_SPLIT
_PAIRS_PER_W = _HALF_B // _ROW_GROUPS
_EDGES_PER_W = N_CONN_K // _EDGE_SPLIT
_LANES = 16


def _sc_spmm_body(xpk_hbm, pk_hbm, ew_hbm, out_hbm, pk_v, ew_v,
                  xpa0, xpa1, xpb0, xpb1, a0, a1, a2, a3,
                  sem_xa, sem_xb, sem_o):
    c = lax.axis_index("c")
    s = lax.axis_index("s")
    wid = s * _NC + c
    q = wid // _ROW_GROUPS
    rgrp = wid % _ROW_GROUPS
    pbase = rgrp * _PAIRS_PER_W

    e0 = q * _EDGES_PER_W
    pltpu.sync_copy(pk_hbm.at[pl.ds(e0, _EDGES_PER_W)], pk_v)
    pltpu.sync_copy(ew_hbm.at[pl.ds(e0, _EDGES_PER_W)], ew_v)

    zeros16 = jnp.zeros((_LANES,), jnp.float32)
    himask = jnp.int32(-65536)

    def zero_acc(acc):
        @plsc.parallel_loop(0, N_TFS_K, step=_LANES, unroll=8)
        def zero_body(j):
            acc[pl.ds(j, _LANES)] = zeros16

    def accumulate_quad(xp0, xp1):
        @plsc.parallel_loop(0, _EDGES_PER_W, step=_LANES, unroll=8)
        def edge_body(e):
            pk = pk_v[pl.ds(e, _LANES)]
            g = lax.shift_right_logical(pk, 11)
            t = lax.bitwise_and(pk, 2047)
            w = ew_v[pl.ds(e, _LANES)]
            for xp, alo, ahi in ((xp0, a0, a1), (xp1, a2, a3)):
                v = plsc.load_gather(xp, [g])
                vlo = plsc.bitcast(lax.shift_left(v, 16), jnp.float32)
                vhi = plsc.bitcast(lax.bitwise_and(v, himask), jnp.float32)
                plsc.addupdate_scatter(alo, [t], vlo * w)
                plsc.addupdate_scatter(ahi, [t], vhi * w)

    def issue_outs(p0):
        pltpu.async_copy(a0, out_hbm.at[q, p0], sem_o)
        pltpu.async_copy(a1, out_hbm.at[q, p0 + _HALF_B], sem_o)
        pltpu.async_copy(a2, out_hbm.at[q, p0 + 1], sem_o)
        pltpu.async_copy(a3, out_hbm.at[q, p0 + 1 + _HALF_B], sem_o)

    def wait_outs(p0):
        pltpu.make_async_copy(a0, out_hbm.at[q, p0], sem_o).wait()
        pltpu.make_async_copy(a1, out_hbm.at[q, p0 + _HALF_B], sem_o).wait()
        pltpu.make_async_copy(a2, out_hbm.at[q, p0 + 1], sem_o).wait()
        pltpu.make_async_copy(a3, out_hbm.at[q, p0 + 1 + _HALF_B], sem_o).wait()

    pltpu.async_copy(xpk_hbm.at[pbase], xpa0, sem_xa)
    pltpu.async_copy(xpk_hbm.at[pbase + 1], xpa1, sem_xa)

    def step_body(i, carry):
        p0 = pbase + 2 * i
        even = lax.rem(i, 2) == 0
        @pl.when(i < _PAIRS_PER_W // 2 - 1)
        def _prefetch():
            @pl.when(even)
            def _pb():
                pltpu.async_copy(xpk_hbm.at[p0 + 2], xpb0, sem_xb)
                pltpu.async_copy(xpk_hbm.at[p0 + 3], xpb1, sem_xb)

            @pl.when(jnp.logical_not(even))
            def _pa():
                pltpu.async_copy(xpk_hbm.at[p0 + 2], xpa0, sem_xa)
                pltpu.async_copy(xpk_hbm.at[p0 + 3], xpa1, sem_xa)

        @pl.when(i > 0)
        def _drain():
            wait_outs(p0 - 2)

        zero_acc(a0)
        zero_acc(a1)
        zero_acc(a2)
        zero_acc(a3)

        @pl.when(even)
        def _runa():
            pltpu.make_async_copy(xpk_hbm.at[p0], xpa0, sem_xa).wait()
            pltpu.make_async_copy(xpk_hbm.at[p0 + 1], xpa1, sem_xa).wait()
            accumulate_quad(xpa0, xpa1)

        @pl.when(jnp.logical_not(even))
        def _runb():
            pltpu.make_async_copy(xpk_hbm.at[p0], xpb0, sem_xb).wait()
            pltpu.make_async_copy(xpk_hbm.at[p0 + 1], xpb1, sem_xb).wait()
            accumulate_quad(xpb0, xpb1)

        issue_outs(p0)
        return carry

    lax.fori_loop(0, _PAIRS_PER_W // 2, step_body, 0)
    wait_outs(pbase + _PAIRS_PER_W - 2)


@jax.jit
def _sc_spmm(xpk, packed, ew):
    mesh = plsc.VectorSubcoreMesh(core_axis_name="c", subcore_axis_name="s",
                                  num_cores=_NC, num_subcores=_NS)
    return pl.kernel(
        _sc_spmm_body,
        out_type=jax.ShapeDtypeStruct((_EDGE_SPLIT, BATCH_K, N_TFS_K),
                                      jnp.float32),
        mesh=mesh,
        scratch_types=[
            pltpu.VMEM((_EDGES_PER_W,), jnp.int32),
            pltpu.VMEM((_EDGES_PER_W,), jnp.float32),
            pltpu.VMEM((N_GENES_K,), jnp.int32),
            pltpu.VMEM((N_GENES_K,), jnp.int32),
            pltpu.VMEM((N_GENES_K,), jnp.int32),
            pltpu.VMEM((N_GENES_K,), jnp.int32),
            pltpu.VMEM((N_TFS_K,), jnp.float32),
            pltpu.VMEM((N_TFS_K,), jnp.float32),
            pltpu.VMEM((N_TFS_K,), jnp.float32),
            pltpu.VMEM((N_TFS_K,), jnp.float32),
            pltpu.SemaphoreType.DMA,
            pltpu.SemaphoreType.DMA,
            pltpu.SemaphoreType.DMA,
        ],
        compiler_params=pltpu.CompilerParams(needs_layout_passes=False),
    )(xpk, packed, ew)


def _prelu(h, a):
    return jnp.maximum(h, 0.0) + a * jnp.minimum(h, 0.0)


_BB = 128
_GB = 2048
_NB = BATCH_K // _BB
_NG = (N_GENES_K + _GB - 1) // _GB


def _mlp_body(p_ref, pe_ref, w1_ref, b1_ref, w2_ref, b2_ref, w3_ref, b3_ref,
              w4_ref, b4_ref, a_ref, out_ref, h_scr):
    j = pl.program_id(1)

    @pl.when(j == 0)
    def _encode():
        tf = (p_ref[0] + p_ref[1]) + (p_ref[2] + p_ref[3])
        a0 = a_ref[0, 0]
        a1 = a_ref[0, 1]
        a2 = a_ref[0, 2]
        a3 = a_ref[0, 3]
        h = _prelu(tf, a0)
        h = lax.dot_general(h.astype(jnp.bfloat16), w1_ref[...],
                            (((1,), (1,)), ((), ())),
                            preferred_element_type=jnp.float32) + b1_ref[...]
        h = _prelu(h, a1)
        h = lax.dot_general(h.astype(jnp.bfloat16), w2_ref[...],
                            (((1,), (1,)), ((), ())),
                            preferred_element_type=jnp.float32) + b2_ref[...]
        h = _prelu(h, a2)
        h = h + pe_ref[...]
        h = lax.dot_general(h.astype(jnp.bfloat16), w3_ref[...],
                            (((1,), (1,)), ((), ())),
                            preferred_element_type=jnp.float32) + b3_ref[...]
        h_scr[...] = _prelu(h, a3).astype(jnp.bfloat16)

    out_ref[...] = lax.dot_general(
        h_scr[...], w4_ref[...], (((1,), (1,)), ((), ())),
        preferred_element_type=jnp.float32) + b4_ref[...]


@jax.jit
def _tc_mlp(partials, pe, W1, b1, W2, b2, W3, b3, W4, b4, a_all):
    grid = (_NB, _NG)
    return pl.pallas_call(
        _mlp_body,
        grid=grid,
        in_specs=[
            pl.BlockSpec((_EDGE_SPLIT, _BB, N_TFS_K), lambda i, j: (0, i, 0)),
            pl.BlockSpec((_BB, 64), lambda i, j: (i, 0)),
            pl.BlockSpec((64, N_TFS_K), lambda i, j: (0, 0)),
            pl.BlockSpec((1, 64), lambda i, j: (0, 0)),
            pl.BlockSpec((64, 64), lambda i, j: (0, 0)),
            pl.BlockSpec((1, 64), lambda i, j: (0, 0)),
            pl.BlockSpec((64, 64), lambda i, j: (0, 0)),
            pl.BlockSpec((1, 64), lambda i, j: (0, 0)),
            pl.BlockSpec((_GB, 64), lambda i, j: (j, 0)),
            pl.BlockSpec((1, _GB), lambda i, j: (0, j)),
            pl.BlockSpec((1, 4), lambda i, j: (0, 0)),
        ],
        out_specs=pl.BlockSpec((_BB, _GB), lambda i, j: (i, j)),
        out_shape=jax.ShapeDtypeStruct((BATCH_K, N_GENES_K), jnp.float32),
        scratch_shapes=[pltpu.VMEM((_BB, 64), jnp.bfloat16)],
    )(partials, pe, W1, b1, W2, b2, W3, b3, W4, b4, a_all)


def kernel(x, pert, gene_indices, tf_indices, edge_weights, pert_table,
           W1, b1, W2, b2, W3, b3, W4, b4, a0, a1, a2, a3):
    packed = (gene_indices.astype(jnp.int32) << 11) | tf_indices.astype(jnp.int32)
    xlo = lax.bitcast_convert_type(x[:_HALF_B].astype(jnp.bfloat16),
                                   jnp.uint16).astype(jnp.uint32)
    xhi = lax.bitcast_convert_type(x[_HALF_B:].astype(jnp.bfloat16),
                                   jnp.uint16).astype(jnp.uint32)
    xpk = lax.bitcast_convert_type((xhi << 16) | xlo, jnp.int32)
    partials = _sc_spmm(xpk, packed, edge_weights)
    pe = jnp.take(pert_table, pert, axis=0)
    a_all = jnp.stack([a0[0], a1[0], a2[0], a3[0]]).reshape(1, 4)
    return _tc_mlp(partials, pe, W1.astype(jnp.bfloat16), b1.reshape(1, 64),
                   W2.astype(jnp.bfloat16), b2.reshape(1, 64),
                   W3.astype(jnp.bfloat16), b3.reshape(1, 64),
                   W4.astype(jnp.bfloat16), b4.reshape(1, 20000), a_all)


# speedup vs baseline: 1.0590x; 1.0590x over previous
import functools

import jax
import jax.numpy as jnp
from jax import lax
from jax.experimental import pallas as pl
from jax.experimental.pallas import tpu as pltpu
from jax.experimental.pallas import tpu_sc as plsc

N_GENES_K = 20000
N_TFS_K = 2048
N_CONN_K = 65536
BATCH_K = 1024
_HALF_B = BATCH_K // 2

_NC = 2
_NS = 16
_NW = _NC * _NS
_EDGE_SPLIT = 4
_ROW_GROUPS = _NW // _EDGE_SPLIT
_PAIRS_PER_W = _HALF_B // _ROW_GROUPS
_EDGES_PER_W = N_CONN_K // _EDGE_SPLIT
_LANES = 16


def _sc_spmm_body(xpk_hbm, pk_hbm, ew_hbm, out_hbm, pk_v, ew_v,
                  xpa0, xpa1, xpb0, xpb1, a0, a1, a2, a3,
                  sem_xa, sem_xb, sem_o):
    c = lax.axis_index("c")
    s = lax.axis_index("s")
    wid = s * _NC + c
    q = wid // _ROW_GROUPS
    rgrp = wid % _ROW_GROUPS
    pbase = rgrp * _PAIRS_PER_W

    e0 = q * _EDGES_PER_W
    pltpu.sync_copy(pk_hbm.at[pl.ds(e0, _EDGES_PER_W)], pk_v)
    pltpu.sync_copy(ew_hbm.at[pl.ds(e0, _EDGES_PER_W)], ew_v)

    zeros16 = jnp.zeros((_LANES,), jnp.float32)
    himask = jnp.int32(-65536)

    def zero_acc(acc):
        @plsc.parallel_loop(0, N_TFS_K, step=_LANES, unroll=8)
        def zero_body(j):
            acc[pl.ds(j, _LANES)] = zeros16

    def accumulate_quad(xp0, xp1):
        @plsc.parallel_loop(0, _EDGES_PER_W, step=_LANES, unroll=8)
        def edge_body(e):
            pk = pk_v[pl.ds(e, _LANES)]
            g = lax.shift_right_logical(pk, 11)
            t = lax.bitwise_and(pk, 2047)
            w = ew_v[pl.ds(e, _LANES)]
            for xp, alo, ahi in ((xp0, a0, a1), (xp1, a2, a3)):
                v = plsc.load_gather(xp, [g])
                vlo = plsc.bitcast(lax.shift_left(v, 16), jnp.float32)
                vhi = plsc.bitcast(lax.bitwise_and(v, himask), jnp.float32)
                plsc.addupdate_scatter(alo, [t], vlo * w)
                plsc.addupdate_scatter(ahi, [t], vhi * w)

    def issue_outs(p0):
        pltpu.async_copy(a0, out_hbm.at[q, p0], sem_o)
        pltpu.async_copy(a1, out_hbm.at[q, p0 + _HALF_B], sem_o)
        pltpu.async_copy(a2, out_hbm.at[q, p0 + 1], sem_o)
        pltpu.async_copy(a3, out_hbm.at[q, p0 + 1 + _HALF_B], sem_o)

    def wait_outs(p0):
        pltpu.make_async_copy(a0, out_hbm.at[q, p0], sem_o).wait()
        pltpu.make_async_copy(a1, out_hbm.at[q, p0 + _HALF_B], sem_o).wait()
        pltpu.make_async_copy(a2, out_hbm.at[q, p0 + 1], sem_o).wait()
        pltpu.make_async_copy(a3, out_hbm.at[q, p0 + 1 + _HALF_B], sem_o).wait()

    pltpu.async_copy(xpk_hbm.at[pbase], xpa0, sem_xa)
    pltpu.async_copy(xpk_hbm.at[pbase + 1], xpa1, sem_xa)

    def step_body(i, carry):
        p0 = pbase + 2 * i
        even = lax.rem(i, 2) == 0
        @pl.when(i < _PAIRS_PER_W // 2 - 1)
        def _prefetch():
            @pl.when(even)
            def _pb():
                pltpu.async_copy(xpk_hbm.at[p0 + 2], xpb0, sem_xb)
                pltpu.async_copy(xpk_hbm.at[p0 + 3], xpb1, sem_xb)

            @pl.when(jnp.logical_not(even))
            def _pa():
                pltpu.async_copy(xpk_hbm.at[p0 + 2], xpa0, sem_xa)
                pltpu.async_copy(xpk_hbm.at[p0 + 3], xpa1, sem_xa)

        @pl.when(i > 0)
        def _drain():
            wait_outs(p0 - 2)

        zero_acc(a0)
        zero_acc(a1)
        zero_acc(a2)
        zero_acc(a3)

        @pl.when(even)
        def _runa():
            pltpu.make_async_copy(xpk_hbm.at[p0], xpa0, sem_xa).wait()
            pltpu.make_async_copy(xpk_hbm.at[p0 + 1], xpa1, sem_xa).wait()
            accumulate_quad(xpa0, xpa1)

        @pl.when(jnp.logical_not(even))
        def _runb():
            pltpu.make_async_copy(xpk_hbm.at[p0], xpb0, sem_xb).wait()
            pltpu.make_async_copy(xpk_hbm.at[p0 + 1], xpb1, sem_xb).wait()
            accumulate_quad(xpb0, xpb1)

        issue_outs(p0)
        return carry

    lax.fori_loop(0, _PAIRS_PER_W // 2, step_body, 0)
    wait_outs(pbase + _PAIRS_PER_W - 2)


@jax.jit
def _sc_spmm(xpk, packed, ew):
    mesh = plsc.VectorSubcoreMesh(core_axis_name="c", subcore_axis_name="s",
                                  num_cores=_NC, num_subcores=_NS)
    return pl.kernel(
        _sc_spmm_body,
        out_type=jax.ShapeDtypeStruct((_EDGE_SPLIT, BATCH_K, N_TFS_K),
                                      jnp.float32),
        mesh=mesh,
        scratch_types=[
            pltpu.VMEM((_EDGES_PER_W,), jnp.int32),
            pltpu.VMEM((_EDGES_PER_W,), jnp.float32),
            pltpu.VMEM((N_GENES_K,), jnp.int32),
            pltpu.VMEM((N_GENES_K,), jnp.int32),
            pltpu.VMEM((N_GENES_K,), jnp.int32),
            pltpu.VMEM((N_GENES_K,), jnp.int32),
            pltpu.VMEM((N_TFS_K,), jnp.float32),
            pltpu.VMEM((N_TFS_K,), jnp.float32),
            pltpu.VMEM((N_TFS_K,), jnp.float32),
            pltpu.VMEM((N_TFS_K,), jnp.float32),
            pltpu.SemaphoreType.DMA,
            pltpu.SemaphoreType.DMA,
            pltpu.SemaphoreType.DMA,
        ],
        compiler_params=pltpu.CompilerParams(needs_layout_passes=False),
    )(xpk, packed, ew)


def _prelu(h, a):
    return jnp.maximum(h, 0.0) + a * jnp.minimum(h, 0.0)


_BB = 256
_GB = 2048
_NB = BATCH_K // _BB
_NG = (N_GENES_K + _GB - 1) // _GB


def _mlp_body(p_ref, pe_ref, w1_ref, b1_ref, w2_ref, b2_ref, w3_ref, b3_ref,
              w4_ref, b4_ref, a_ref, out_ref, h_scr):
    j = pl.program_id(1)

    @pl.when(j == 0)
    def _encode():
        tf = (p_ref[0] + p_ref[1]) + (p_ref[2] + p_ref[3])
        a0 = a_ref[0, 0]
        a1 = a_ref[0, 1]
        a2 = a_ref[0, 2]
        a3 = a_ref[0, 3]
        h = _prelu(tf, a0)
        h = lax.dot_general(h.astype(jnp.bfloat16), w1_ref[...],
                            (((1,), (1,)), ((), ())),
                            preferred_element_type=jnp.float32) + b1_ref[...]
        h = _prelu(h, a1)
        h = lax.dot_general(h.astype(jnp.bfloat16), w2_ref[...],
                            (((1,), (1,)), ((), ())),
                            preferred_element_type=jnp.float32) + b2_ref[...]
        h = _prelu(h, a2)
        h = h + pe_ref[...]
        h = lax.dot_general(h.astype(jnp.bfloat16), w3_ref[...],
                            (((1,), (1,)), ((), ())),
                            preferred_element_type=jnp.float32) + b3_ref[...]
        h_scr[...] = _prelu(h, a3).astype(jnp.bfloat16)

    out_ref[...] = lax.dot_general(
        h_scr[...], w4_ref[...], (((1,), (1,)), ((), ())),
        preferred_element_type=jnp.float32) + b4_ref[...]


@jax.jit
def _tc_mlp(partials, pe, W1, b1, W2, b2, W3, b3, W4, b4, a_all):
    grid = (_NB, _NG)
    return pl.pallas_call(
        _mlp_body,
        grid=grid,
        in_specs=[
            pl.BlockSpec((_EDGE_SPLIT, _BB, N_TFS_K), lambda i, j: (0, i, 0)),
            pl.BlockSpec((_BB, 64), lambda i, j: (i, 0)),
            pl.BlockSpec((64, N_TFS_K), lambda i, j: (0, 0)),
            pl.BlockSpec((1, 64), lambda i, j: (0, 0)),
            pl.BlockSpec((64, 64), lambda i, j: (0, 0)),
            pl.BlockSpec((1, 64), lambda i, j: (0, 0)),
            pl.BlockSpec((64, 64), lambda i, j: (0, 0)),
            pl.BlockSpec((1, 64), lambda i, j: (0, 0)),
            pl.BlockSpec((_GB, 64), lambda i, j: (j, 0)),
            pl.BlockSpec((1, _GB), lambda i, j: (0, j)),
            pl.BlockSpec((1, 4), lambda i, j: (0, 0)),
        ],
        out_specs=pl.BlockSpec((_BB, _GB), lambda i, j: (i, j)),
        out_shape=jax.ShapeDtypeStruct((BATCH_K, N_GENES_K), jnp.float32),
        scratch_shapes=[pltpu.VMEM((_BB, 64), jnp.bfloat16)],
    )(partials, pe, W1, b1, W2, b2, W3, b3, W4, b4, a_all)


def kernel(x, pert, gene_indices, tf_indices, edge_weights, pert_table,
           W1, b1, W2, b2, W3, b3, W4, b4, a0, a1, a2, a3):
    packed = (gene_indices.astype(jnp.int32) << 11) | tf_indices.astype(jnp.int32)
    xlo = lax.bitcast_convert_type(x[:_HALF_B].astype(jnp.bfloat16),
                                   jnp.uint16).astype(jnp.uint32)
    xhi = lax.bitcast_convert_type(x[_HALF_B:].astype(jnp.bfloat16),
                                   jnp.uint16).astype(jnp.uint32)
    xpk = lax.bitcast_convert_type((xhi << 16) | xlo, jnp.int32)
    partials = _sc_spmm(xpk, packed, edge_weights)
    pe = jnp.take(pert_table, pert, axis=0)
    a_all = jnp.stack([a0[0], a1[0], a2[0], a3[0]]).reshape(1, 4)
    return _tc_mlp(partials, pe, W1.astype(jnp.bfloat16), b1.reshape(1, 64),
                   W2.astype(jnp.bfloat16), b2.reshape(1, 64),
                   W3.astype(jnp.bfloat16), b3.reshape(1, 64),
                   W4.astype(jnp.bfloat16), b4.reshape(1, 20000), a_all)


# speedup vs baseline: 1.0781x; 1.0181x over previous
import jax
import jax.numpy as jnp
from jax import lax
from jax.experimental import pallas as pl
from jax.experimental.pallas import tpu as pltpu
from jax.experimental.pallas import tpu_sc as plsc

N_GENES_K = 20000
N_TFS_K = 2048
N_CONN_K = 65536
BATCH_K = 1024
_HALF_B = BATCH_K // 2

_NC = 2
_NS = 16
_NW = _NC * _NS
_EDGE_SPLIT = 4
_ROW_GROUPS = _NW // _EDGE_SPLIT
_PAIRS_PER_W = _HALF_B // _ROW_GROUPS
_EDGES_PER_W = N_CONN_K // _EDGE_SPLIT
_LANES = 16


def _sc_spmm_body(xpk_hbm, pk_hbm, ew_hbm, out_hbm, pk_v, ew_v,
                  xpa0, xpa1, xpb0, xpb1, a0, a1, a2, a3,
                  sem_xa, sem_xb, sem_o):
    c = lax.axis_index("c")
    s = lax.axis_index("s")
    wid = s * _NC + c
    q = wid // _ROW_GROUPS
    rgrp = wid % _ROW_GROUPS
    pbase = rgrp * _PAIRS_PER_W

    e0 = q * _EDGES_PER_W
    pltpu.sync_copy(pk_hbm.at[pl.ds(e0, _EDGES_PER_W)], pk_v)
    pltpu.sync_copy(ew_hbm.at[pl.ds(e0, _EDGES_PER_W)], ew_v)

    zeros16 = jnp.zeros((_LANES,), jnp.float32)
    himask = jnp.int32(-65536)

    def zero_acc(acc):
        @plsc.parallel_loop(0, N_TFS_K, step=_LANES, unroll=8)
        def zero_body(j):
            acc[pl.ds(j, _LANES)] = zeros16

    def accumulate_quad(xp0, xp1):
        @plsc.parallel_loop(0, _EDGES_PER_W, step=_LANES, unroll=8)
        def edge_body(e):
            pk = pk_v[pl.ds(e, _LANES)]
            g = lax.shift_right_logical(pk, 11)
            t = lax.bitwise_and(pk, 2047)
            w = ew_v[pl.ds(e, _LANES)]
            for xp, alo, ahi in ((xp0, a0, a1), (xp1, a2, a3)):
                v = plsc.load_gather(xp, [g])
                vlo = plsc.bitcast(lax.shift_left(v, 16), jnp.float32)
                vhi = plsc.bitcast(lax.bitwise_and(v, himask), jnp.float32)
                plsc.addupdate_scatter(alo, [t], vlo * w)
                plsc.addupdate_scatter(ahi, [t], vhi * w)

    def issue_outs(p0):
        pltpu.async_copy(a0, out_hbm.at[q, p0], sem_o)
        pltpu.async_copy(a1, out_hbm.at[q, p0 + _HALF_B], sem_o)
        pltpu.async_copy(a2, out_hbm.at[q, p0 + 1], sem_o)
        pltpu.async_copy(a3, out_hbm.at[q, p0 + 1 + _HALF_B], sem_o)

    def wait_outs(p0):
        pltpu.make_async_copy(a0, out_hbm.at[q, p0], sem_o).wait()
        pltpu.make_async_copy(a1, out_hbm.at[q, p0 + _HALF_B], sem_o).wait()
        pltpu.make_async_copy(a2, out_hbm.at[q, p0 + 1], sem_o).wait()
        pltpu.make_async_copy(a3, out_hbm.at[q, p0 + 1 + _HALF_B], sem_o).wait()

    pltpu.async_copy(xpk_hbm.at[pbase], xpa0, sem_xa)
    pltpu.async_copy(xpk_hbm.at[pbase + 1], xpa1, sem_xa)

    def step_body(i, carry):
        p0 = pbase + 2 * i
        even = lax.rem(i, 2) == 0
        @pl.when(i < _PAIRS_PER_W // 2 - 1)
        def _prefetch():
            @pl.when(even)
            def _pb():
                pltpu.async_copy(xpk_hbm.at[p0 + 2], xpb0, sem_xb)
                pltpu.async_copy(xpk_hbm.at[p0 + 3], xpb1, sem_xb)

            @pl.when(jnp.logical_not(even))
            def _pa():
                pltpu.async_copy(xpk_hbm.at[p0 + 2], xpa0, sem_xa)
                pltpu.async_copy(xpk_hbm.at[p0 + 3], xpa1, sem_xa)

        @pl.when(i > 0)
        def _drain():
            wait_outs(p0 - 2)

        zero_acc(a0)
        zero_acc(a1)
        zero_acc(a2)
        zero_acc(a3)

        @pl.when(even)
        def _runa():
            pltpu.make_async_copy(xpk_hbm.at[p0], xpa0, sem_xa).wait()
            pltpu.make_async_copy(xpk_hbm.at[p0 + 1], xpa1, sem_xa).wait()
            accumulate_quad(xpa0, xpa1)

        @pl.when(jnp.logical_not(even))
        def _runb():
            pltpu.make_async_copy(xpk_hbm.at[p0], xpb0, sem_xb).wait()
            pltpu.make_async_copy(xpk_hbm.at[p0 + 1], xpb1, sem_xb).wait()
            accumulate_quad(xpb0, xpb1)

        issue_outs(p0)
        return carry

    lax.fori_loop(0, _PAIRS_PER_W // 2, step_body, 0)
    wait_outs(pbase + _PAIRS_PER_W - 2)


@jax.jit
def _sc_spmm(xpk, packed, ew):
    mesh = plsc.VectorSubcoreMesh(core_axis_name="c", subcore_axis_name="s",
                                  num_cores=_NC, num_subcores=_NS)
    return pl.kernel(
        _sc_spmm_body,
        out_type=jax.ShapeDtypeStruct((_EDGE_SPLIT, BATCH_K, N_TFS_K),
                                      jnp.float32),
        mesh=mesh,
        scratch_types=[
            pltpu.VMEM((_EDGES_PER_W,), jnp.int32),
            pltpu.VMEM((_EDGES_PER_W,), jnp.float32),
            pltpu.VMEM((N_GENES_K,), jnp.int32),
            pltpu.VMEM((N_GENES_K,), jnp.int32),
            pltpu.VMEM((N_GENES_K,), jnp.int32),
            pltpu.VMEM((N_GENES_K,), jnp.int32),
            pltpu.VMEM((N_TFS_K,), jnp.float32),
            pltpu.VMEM((N_TFS_K,), jnp.float32),
            pltpu.VMEM((N_TFS_K,), jnp.float32),
            pltpu.VMEM((N_TFS_K,), jnp.float32),
            pltpu.SemaphoreType.DMA,
            pltpu.SemaphoreType.DMA,
            pltpu.SemaphoreType.DMA,
        ],
        compiler_params=pltpu.CompilerParams(needs_layout_passes=False),
    )(xpk, packed, ew)


def _prelu(h, a):
    return jnp.maximum(h, 0.0) + a * jnp.minimum(h, 0.0)


_BB = 256
_GB = 4096
_NB = BATCH_K // _BB
_NG = (N_GENES_K + _GB - 1) // _GB


def _mlp_body(p_ref, pe_ref, w1_ref, b1_ref, w2_ref, b2_ref, w3_ref, b3_ref,
              w4_ref, b4_ref, a_ref, out_ref, h_scr):
    j = pl.program_id(1)

    @pl.when(j == 0)
    def _encode():
        tf = (p_ref[0] + p_ref[1]) + (p_ref[2] + p_ref[3])
        a0 = a_ref[0, 0]
        a1 = a_ref[0, 1]
        a2 = a_ref[0, 2]
        a3 = a_ref[0, 3]
        h = _prelu(tf, a0)
        h = lax.dot_general(h.astype(jnp.bfloat16), w1_ref[...],
                            (((1,), (1,)), ((), ())),
                            preferred_element_type=jnp.float32) + b1_ref[...]
        h = _prelu(h, a1)
        h = lax.dot_general(h.astype(jnp.bfloat16), w2_ref[...],
                            (((1,), (1,)), ((), ())),
                            preferred_element_type=jnp.float32) + b2_ref[...]
        h = _prelu(h, a2)
        h = h + pe_ref[...]
        h = lax.dot_general(h.astype(jnp.bfloat16), w3_ref[...],
                            (((1,), (1,)), ((), ())),
                            preferred_element_type=jnp.float32) + b3_ref[...]
        h_scr[...] = _prelu(h, a3).astype(jnp.bfloat16)

    out_ref[...] = lax.dot_general(
        h_scr[...], w4_ref[...], (((1,), (1,)), ((), ())),
        preferred_element_type=jnp.float32) + b4_ref[...]


@jax.jit
def _tc_mlp(partials, pe, W1, b1, W2, b2, W3, b3, W4, b4, a_all):
    grid = (_NB, _NG)
    return pl.pallas_call(
        _mlp_body,
        grid=grid,
        in_specs=[
            pl.BlockSpec((_EDGE_SPLIT, _BB, N_TFS_K), lambda i, j: (0, i, 0)),
            pl.BlockSpec((_BB, 64), lambda i, j: (i, 0)),
            pl.BlockSpec((64, N_TFS_K), lambda i, j: (0, 0)),
            pl.BlockSpec((1, 64), lambda i, j: (0, 0)),
            pl.BlockSpec((64, 64), lambda i, j: (0, 0)),
            pl.BlockSpec((1, 64), lambda i, j: (0, 0)),
            pl.BlockSpec((64, 64), lambda i, j: (0, 0)),
            pl.BlockSpec((1, 64), lambda i, j: (0, 0)),
            pl.BlockSpec((_GB, 64), lambda i, j: (j, 0)),
            pl.BlockSpec((1, _GB), lambda i, j: (0, j)),
            pl.BlockSpec((1, 4), lambda i, j: (0, 0)),
        ],
        out_specs=pl.BlockSpec((_BB, _GB), lambda i, j: (i, j)),
        out_shape=jax.ShapeDtypeStruct((BATCH_K, N_GENES_K), jnp.float32),
        scratch_shapes=[pltpu.VMEM((_BB, 64), jnp.bfloat16)],
    )(partials, pe, W1, b1, W2, b2, W3, b3, W4, b4, a_all)


def kernel(x, pert, gene_indices, tf_indices, edge_weights, pert_table,
           W1, b1, W2, b2, W3, b3, W4, b4, a0, a1, a2, a3):
    packed = (gene_indices.astype(jnp.int32) << 11) | tf_indices.astype(jnp.int32)
    xlo = lax.bitcast_convert_type(x[:_HALF_B].astype(jnp.bfloat16),
                                   jnp.uint16).astype(jnp.uint32)
    xhi = lax.bitcast_convert_type(x[_HALF_B:].astype(jnp.bfloat16),
                                   jnp.uint16).astype(jnp.uint32)
    xpk = lax.bitcast_convert_type((xhi << 16) | xlo, jnp.int32)
    partials = _sc_spmm(xpk, packed, edge_weights)
    pe = jnp.take(pert_table, pert, axis=0)
    a_all = jnp.stack([a0[0], a1[0], a2[0], a3[0]]).reshape(1, 4)
    return _tc_mlp(partials, pe, W1.astype(jnp.bfloat16), b1.reshape(1, 64),
                   W2.astype(jnp.bfloat16), b2.reshape(1, 64),
                   W3.astype(jnp.bfloat16), b3.reshape(1, 64),
                   W4.astype(jnp.bfloat16), b4.reshape(1, 20000), a_all)
